# baseline (speedup 1.0000x reference)
import functools

import jax
import jax.numpy as jnp
from jax import lax
from jax.experimental import pallas as pl
from jax.experimental.pallas import tpu as pltpu
from jax.experimental.pallas import tpu_sc as plsc

NC = 2
NS = 16
NW = NC * NS
BATCH = 80
BLK = 1280


def _sc_agg(h, src, dst):
  npad, d = h.shape
  e = src.shape[0]
  per_w = e // NW
  nb = per_w // BATCH
  assert per_w * NW == e and nb * BATCH == per_w
  rpt = npad // NS
  nchunk = rpt // BATCH
  assert rpt * NS == npad and nchunk * BATCH == rpt

  mesh = plsc.VectorSubcoreMesh(core_axis_name="c", subcore_axis_name="s")

  @functools.partial(
      pl.kernel,
      out_type=[
          jax.ShapeDtypeStruct((NC * npad, d), jnp.float32),
          jax.ShapeDtypeStruct((NC * npad,), jnp.float32),
      ],
      mesh=mesh,
      scratch_types=[
          pltpu.VMEM((BATCH,), jnp.int32),
          pltpu.VMEM((BATCH,), jnp.int32),
          pltpu.VMEM((BATCH,), jnp.int32),
          pltpu.VMEM((BATCH,), jnp.int32),
          pltpu.VMEM((BATCH, d), jnp.float32),
          pltpu.VMEM((BATCH, d), jnp.float32),
          pltpu.VMEM((BATCH,), jnp.float32),
          pltpu.VMEM_SHARED((npad, d), jnp.float32),
          pltpu.VMEM_SHARED((npad,), jnp.float32),
          pltpu.SemaphoreType.DMA,
          pltpu.SemaphoreType.DMA,
          pltpu.SemaphoreType.DMA,
          pltpu.SemaphoreType.DMA,
      ],
  )
  def k(h_hbm, src_hbm, dst_hbm, agg_out, deg_out,
        src_a, dst_a, src_b, dst_b, rows_a, rows_b, ones_v,
        agg_sh, deg_sh, sem_a, sem_b, sem_sa, sem_sb):
    c = lax.axis_index("c")
    s = lax.axis_index("s")
    wid = s * NC + c
    zed = jnp.zeros((16,), jnp.float32)

    def zrow(i, carry):
      for j in range(d // 16):
        rows_a[i, pl.ds(j * 16, 16)] = zed
      return carry

    lax.fori_loop(0, BATCH, zrow, 0)
    for j in range(BATCH // 16):
      ones_v[pl.ds(j * 16, 16)] = zed
    for j in range(nchunk):
      off = pl.multiple_of(s * rpt + j * BATCH, 8)
      pltpu.sync_copy(rows_a, agg_sh.at[pl.ds(off, BATCH)])
      pltpu.sync_copy(ones_v, deg_sh.at[pl.ds(off, BATCH)])
    one = jnp.ones((16,), jnp.float32)
    for j in range(BATCH // 16):
      ones_v[pl.ds(j * 16, 16)] = one
    plsc.subcore_barrier()

    base = wid * per_w
    assert nb % 2 == 0 and nb >= 4

    def ldidx(bi, sref, dref):
      off = pl.multiple_of(base + bi * BATCH, 8)
      pltpu.sync_copy(src_hbm.at[pl.ds(off, BATCH)], sref)
      pltpu.sync_copy(dst_hbm.at[pl.ds(off, BATCH)], dref)

    ldidx(0, src_a, dst_a)
    pltpu.async_copy(h_hbm.at[src_a], rows_a, sem_a)
    ldidx(1, src_b, dst_b)
    pltpu.async_copy(h_hbm.at[src_b], rows_b, sem_b)

    def scat(rows, dstr, sem):
      pltpu.async_copy(rows, agg_sh.at[dstr], sem, add=True)
      pltpu.async_copy(ones_v, deg_sh.at[dstr], sem, add=True)

    def scat_wait(rows, dstr, sem):
      pltpu.make_async_copy(rows, agg_sh.at[dstr], sem).wait()
      pltpu.make_async_copy(ones_v, deg_sh.at[dstr], sem).wait()

    def body(i, carry):
      pltpu.make_async_copy(h_hbm.at[src_a], rows_a, sem_a).wait()
      scat(rows_a, dst_a, sem_sa)
      pltpu.make_async_copy(h_hbm.at[src_b], rows_b, sem_b).wait()
      scat(rows_b, dst_b, sem_sb)
      scat_wait(rows_a, dst_a, sem_sa)
      ldidx(2 * i + 2, src_a, dst_a)
      pltpu.async_copy(h_hbm.at[src_a], rows_a, sem_a)
      scat_wait(rows_b, dst_b, sem_sb)
      ldidx(2 * i + 3, src_b, dst_b)
      pltpu.async_copy(h_hbm.at[src_b], rows_b, sem_b)
      return carry

    lax.fori_loop(0, nb // 2 - 1, body, 0)
    pltpu.make_async_copy(h_hbm.at[src_a], rows_a, sem_a).wait()
    scat(rows_a, dst_a, sem_sa)
    pltpu.make_async_copy(h_hbm.at[src_b], rows_b, sem_b).wait()
    scat(rows_b, dst_b, sem_sb)
    scat_wait(rows_a, dst_a, sem_sa)
    scat_wait(rows_b, dst_b, sem_sb)

    plsc.subcore_barrier()
    for j in range(nchunk):
      off = pl.multiple_of(s * rpt + j * BATCH, 8)
      oo = pl.multiple_of(c * npad + s * rpt + j * BATCH, 8)
      pltpu.sync_copy(agg_sh.at[pl.ds(off, BATCH)], rows_a)
      pltpu.sync_copy(rows_a, agg_out.at[pl.ds(oo, BATCH)])
      pltpu.sync_copy(deg_sh.at[pl.ds(off, BATCH)], ones_v)
      pltpu.sync_copy(ones_v, deg_out.at[pl.ds(oo, BATCH)])

  return k(h, src, dst)


def _tc_init(features, w, b):
  npad, kdim = features.shape
  d = w.shape[1]

  def body(x_ref, w_ref, b_ref, o_ref):
    o = jnp.dot(x_ref[...], w_ref[...], preferred_element_type=jnp.float32)
    o_ref[...] = jnp.maximum(o + b_ref[...], 0.0)

  return pl.pallas_call(
      body,
      grid=(npad // BLK,),
      in_specs=[
          pl.BlockSpec((BLK, kdim), lambda i: (i, 0)),
          pl.BlockSpec((kdim, d), lambda i: (0, 0)),
          pl.BlockSpec((1, d), lambda i: (0, 0)),
      ],
      out_specs=pl.BlockSpec((BLK, d), lambda i: (i, 0)),
      out_shape=jax.ShapeDtypeStruct((npad, d), jnp.float32),
  )(features, w, b.reshape(1, d))


def _tc_combine(h, agg2, deg4, ws, bs, wn, bn, act):
  npad, d = h.shape
  nblk = npad // BLK

  def body(h_ref, a_ref, g_ref, ws_ref, bs_ref, wn_ref, bn_ref, o_ref):
    a = a_ref[0] + a_ref[1]
    deg = g_ref[0, 0, 0] + g_ref[1, 0, 0]
    hn = a * (1.0 / jnp.maximum(deg, 1.0))[:, None]
    o = (jnp.dot(h_ref[...], ws_ref[...], preferred_element_type=jnp.float32)
         + jnp.dot(hn, wn_ref[...], preferred_element_type=jnp.float32)
         + bs_ref[...] + bn_ref[...])
    if act:
      o = jnp.maximum(o, 0.0)
    o_ref[...] = o

  return pl.pallas_call(
      body,
      grid=(nblk,),
      in_specs=[
          pl.BlockSpec((BLK, d), lambda i: (i, 0)),
          pl.BlockSpec((NC, BLK, d), lambda i: (0, i, 0)),
          pl.BlockSpec((NC, 1, 1, BLK), lambda i: (0, i, 0, 0)),
          pl.BlockSpec((d, d), lambda i: (0, 0)),
          pl.BlockSpec((1, d), lambda i: (0, 0)),
          pl.BlockSpec((d, d), lambda i: (0, 0)),
          pl.BlockSpec((1, d), lambda i: (0, 0)),
      ],
      out_specs=pl.BlockSpec((BLK, d), lambda i: (i, 0)),
      out_shape=jax.ShapeDtypeStruct((npad, d), jnp.float32),
  )(h, agg2, deg4, ws, bs.reshape(1, d), wn, bn.reshape(1, d))


def kernel(features, edge_index0, edge_index1, fc_init_W, fc_init_b,
           fc_self_W, fc_self_b, fc_neigh_W, fc_neigh_b):
  n = features.shape[0]
  d = fc_init_W.shape[1]
  npad = -(-n // (NS * BATCH)) * (NS * BATCH)

  fpad = jnp.pad(features, ((0, npad - n), (0, 0)))
  h0 = _tc_init(fpad, fc_init_W, fc_init_b)

  src0 = edge_index0[0].astype(jnp.int32)
  dst0 = edge_index0[1].astype(jnp.int32)
  agg0, deg0 = _sc_agg(h0, src0, dst0)
  h1 = _tc_combine(h0, agg0.reshape(NC, npad, d),
                   deg0.reshape(NC, npad // BLK, 1, BLK),
                   fc_self_W, fc_self_b, fc_neigh_W, fc_neigh_b, True)

  src1 = edge_index1[0].astype(jnp.int32)
  dst1 = edge_index1[1].astype(jnp.int32)
  agg1, deg1 = _sc_agg(h1, src1, dst1)
  h2 = _tc_combine(h1, agg1.reshape(NC, npad, d),
                   deg1.reshape(NC, npad // BLK, 1, BLK),
                   fc_self_W, fc_self_b, fc_neigh_W, fc_neigh_b, False)
  return h2[:n]


# speedup vs baseline: 1.0491x; 1.0491x over previous
import functools

import jax
import jax.numpy as jnp
from jax import lax
from jax.experimental import pallas as pl
from jax.experimental.pallas import tpu as pltpu
from jax.experimental.pallas import tpu_sc as plsc

NC = 2
NS = 16
NW = NC * NS
BATCH = 80
NSLOT = 4
BLK = 1280


def _sc_agg(h, src, dst):
  npad, d = h.shape
  e = src.shape[0]
  per_w = e // NW
  nb = per_w // BATCH
  assert per_w * NW == e and nb * BATCH == per_w
  rpt = npad // NS
  nchunk = rpt // BATCH
  assert rpt * NS == npad and nchunk * BATCH == rpt

  mesh = plsc.VectorSubcoreMesh(core_axis_name="c", subcore_axis_name="s")

  @functools.partial(
      pl.kernel,
      out_type=[
          jax.ShapeDtypeStruct((NC * npad, d), jnp.float32),
          jax.ShapeDtypeStruct((NC * npad,), jnp.float32),
      ],
      mesh=mesh,
      scratch_types=(
          [pltpu.VMEM((BATCH,), jnp.int32) for _ in range(2 * NSLOT)]
          + [pltpu.VMEM((BATCH, d), jnp.float32) for _ in range(NSLOT)]
          + [pltpu.VMEM((BATCH,), jnp.float32),
             pltpu.VMEM_SHARED((npad, d), jnp.float32),
             pltpu.VMEM_SHARED((npad,), jnp.float32)]
          + [pltpu.SemaphoreType.DMA for _ in range(2 * NSLOT)]
      ),
  )
  def k(h_hbm, src_hbm, dst_hbm, agg_out, deg_out, *refs):
    srcs = refs[0:NSLOT]
    dsts = refs[NSLOT:2 * NSLOT]
    rows = refs[2 * NSLOT:3 * NSLOT]
    ones_v = refs[3 * NSLOT]
    agg_sh = refs[3 * NSLOT + 1]
    deg_sh = refs[3 * NSLOT + 2]
    gsem = refs[3 * NSLOT + 3:4 * NSLOT + 3]
    ssem = refs[4 * NSLOT + 3:5 * NSLOT + 3]
    rows_a = rows[0]
    c = lax.axis_index("c")
    s = lax.axis_index("s")
    wid = s * NC + c
    zed = jnp.zeros((16,), jnp.float32)

    def zrow(i, carry):
      for j in range(d // 16):
        rows_a[i, pl.ds(j * 16, 16)] = zed
      return carry

    lax.fori_loop(0, BATCH, zrow, 0)
    for j in range(BATCH // 16):
      ones_v[pl.ds(j * 16, 16)] = zed
    for j in range(nchunk):
      off = pl.multiple_of(s * rpt + j * BATCH, 8)
      pltpu.sync_copy(rows_a, agg_sh.at[pl.ds(off, BATCH)])
      pltpu.sync_copy(ones_v, deg_sh.at[pl.ds(off, BATCH)])
    one = jnp.ones((16,), jnp.float32)
    for j in range(BATCH // 16):
      ones_v[pl.ds(j * 16, 16)] = one
    plsc.subcore_barrier()

    base = wid * per_w
    ngroup = nb // NSLOT
    tail = nb - ngroup * NSLOT
    assert ngroup >= 2

    def ldidx(bi, sref, dref):
      off = pl.multiple_of(base + bi * BATCH, 8)
      pltpu.sync_copy(src_hbm.at[pl.ds(off, BATCH)], sref)
      pltpu.sync_copy(dst_hbm.at[pl.ds(off, BATCH)], dref)

    def gwait(j):
      pltpu.make_async_copy(h_hbm.at[srcs[j]], rows[j], gsem[j]).wait()

    def scat(j):
      pltpu.async_copy(rows[j], agg_sh.at[dsts[j]], ssem[j], add=True)
      pltpu.async_copy(ones_v, deg_sh.at[dsts[j]], ssem[j], add=True)

    def scat_wait(j):
      pltpu.make_async_copy(rows[j], agg_sh.at[dsts[j]], ssem[j]).wait()
      pltpu.make_async_copy(ones_v, deg_sh.at[dsts[j]], ssem[j]).wait()

    for j in range(NSLOT):
      ldidx(j, srcs[j], dsts[j])
      pltpu.async_copy(h_hbm.at[srcs[j]], rows[j], gsem[j])

    def body(i, carry):
      for j in range(NSLOT):
        gwait(j)
        scat(j)
        scat_wait(j)
        ldidx(NSLOT * i + j + NSLOT, srcs[j], dsts[j])
        pltpu.async_copy(h_hbm.at[srcs[j]], rows[j], gsem[j])
      return carry

    lax.fori_loop(0, ngroup - 1, body, 0)
    for j in range(NSLOT):
      gwait(j)
      scat(j)
    for j in range(NSLOT):
      scat_wait(j)
    for t in range(tail):
      ldidx(ngroup * NSLOT + t, srcs[0], dsts[0])
      pltpu.async_copy(h_hbm.at[srcs[0]], rows[0], gsem[0])
      gwait(0)
      scat(0)
      scat_wait(0)

    plsc.subcore_barrier()
    for j in range(nchunk):
      off = pl.multiple_of(s * rpt + j * BATCH, 8)
      oo = pl.multiple_of(c * npad + s * rpt + j * BATCH, 8)
      pltpu.sync_copy(agg_sh.at[pl.ds(off, BATCH)], rows_a)
      pltpu.sync_copy(rows_a, agg_out.at[pl.ds(oo, BATCH)])
      pltpu.sync_copy(deg_sh.at[pl.ds(off, BATCH)], ones_v)
      pltpu.sync_copy(ones_v, deg_out.at[pl.ds(oo, BATCH)])

  return k(h, src, dst)


def _tc_init(features, w, b):
  npad, kdim = features.shape
  d = w.shape[1]

  def body(x_ref, w_ref, b_ref, o_ref):
    o = jnp.dot(x_ref[...], w_ref[...], preferred_element_type=jnp.float32)
    o_ref[...] = jnp.maximum(o + b_ref[...], 0.0)

  return pl.pallas_call(
      body,
      grid=(npad // BLK,),
      in_specs=[
          pl.BlockSpec((BLK, kdim), lambda i: (i, 0)),
          pl.BlockSpec((kdim, d), lambda i: (0, 0)),
          pl.BlockSpec((1, d), lambda i: (0, 0)),
      ],
      out_specs=pl.BlockSpec((BLK, d), lambda i: (i, 0)),
      out_shape=jax.ShapeDtypeStruct((npad, d), jnp.float32),
  )(features, w, b.reshape(1, d))


def _tc_combine(h, agg2, deg4, ws, bs, wn, bn, act):
  npad, d = h.shape
  nblk = npad // BLK

  def body(h_ref, a_ref, g_ref, ws_ref, bs_ref, wn_ref, bn_ref, o_ref):
    a = a_ref[0] + a_ref[1]
    deg = g_ref[0, 0, 0] + g_ref[1, 0, 0]
    hn = a * (1.0 / jnp.maximum(deg, 1.0))[:, None]
    o = (jnp.dot(h_ref[...], ws_ref[...], preferred_element_type=jnp.float32)
         + jnp.dot(hn, wn_ref[...], preferred_element_type=jnp.float32)
         + bs_ref[...] + bn_ref[...])
    if act:
      o = jnp.maximum(o, 0.0)
    o_ref[...] = o

  return pl.pallas_call(
      body,
      grid=(nblk,),
      in_specs=[
          pl.BlockSpec((BLK, d), lambda i: (i, 0)),
          pl.BlockSpec((NC, BLK, d), lambda i: (0, i, 0)),
          pl.BlockSpec((NC, 1, 1, BLK), lambda i: (0, i, 0, 0)),
          pl.BlockSpec((d, d), lambda i: (0, 0)),
          pl.BlockSpec((1, d), lambda i: (0, 0)),
          pl.BlockSpec((d, d), lambda i: (0, 0)),
          pl.BlockSpec((1, d), lambda i: (0, 0)),
      ],
      out_specs=pl.BlockSpec((BLK, d), lambda i: (i, 0)),
      out_shape=jax.ShapeDtypeStruct((npad, d), jnp.float32),
  )(h, agg2, deg4, ws, bs.reshape(1, d), wn, bn.reshape(1, d))


def kernel(features, edge_index0, edge_index1, fc_init_W, fc_init_b,
           fc_self_W, fc_self_b, fc_neigh_W, fc_neigh_b):
  n = features.shape[0]
  d = fc_init_W.shape[1]
  npad = -(-n // (NS * BATCH)) * (NS * BATCH)

  fpad = jnp.pad(features, ((0, npad - n), (0, 0)))
  h0 = _tc_init(fpad, fc_init_W, fc_init_b)

  src0 = edge_index0[0].astype(jnp.int32)
  dst0 = edge_index0[1].astype(jnp.int32)
  agg0, deg0 = _sc_agg(h0, src0, dst0)
  h1 = _tc_combine(h0, agg0.reshape(NC, npad, d),
                   deg0.reshape(NC, npad // BLK, 1, BLK),
                   fc_self_W, fc_self_b, fc_neigh_W, fc_neigh_b, True)

  src1 = edge_index1[0].astype(jnp.int32)
  dst1 = edge_index1[1].astype(jnp.int32)
  agg1, deg1 = _sc_agg(h1, src1, dst1)
  h2 = _tc_combine(h1, agg1.reshape(NC, npad, d),
                   deg1.reshape(NC, npad // BLK, 1, BLK),
                   fc_self_W, fc_self_b, fc_neigh_W, fc_neigh_b, False)
  return h2[:n]
